# baseline (speedup 1.0000x reference)
import functools

import jax
import jax.numpy as jnp
import numpy as np
from jax import lax
from jax.experimental import pallas as pl
from jax.experimental.pallas import tpu as pltpu
from jax.experimental.pallas import tpu_sc as plsc

N_NODES = 10000
N_EDGES = 320000
D_MODEL = 128
N_HEAD = 8

_NC = 2
_NS = 16
_NW = _NC * _NS
_CH = 40
_CPT = 256
_EPAD = _NW * _CPT * _CH
_NBLK = _CPT // 2
_WIN = 8
_NT = _CPT // _WIN
_D2 = D_MODEL + 16
_NPAD = 10240
_RPT = _NPAD // _NS
_ZB = 8

_PERM = np.zeros(D_MODEL, np.int32)
_SPERM = np.zeros(D_MODEL, np.int32)
for _m in range(8):
    _g, _par = divmod(_m, 2)
    for _l in range(16):
        _h = _l if _l < 8 else 15 - _l
        _d = 2 * _m if _l < 8 else 2 * _m + 1
        _oc = 16 * _h + _d
        _PERM[32 * _g + 2 * _l + _par] = _oc
        _SPERM[16 * _m + _l] = _oc
_INVSPERM = np.argsort(_SPERM)


def _qkv_body(emb_ref, qw_ref, kw_ref, vw_ref, qo_ref, kvo_ref):
    e = emb_ref[...]
    hi = lax.Precision.HIGHEST
    qo_ref[...] = jnp.dot(e, qw_ref[...], precision=hi).astype(jnp.bfloat16)
    kvo_ref[:, pl.ds(0, D_MODEL)] = jnp.dot(
        e, kw_ref[...], precision=hi).astype(jnp.bfloat16)
    kvo_ref[:, pl.ds(D_MODEL, D_MODEL)] = jnp.dot(
        e, vw_ref[...], precision=hi).astype(jnp.bfloat16)


def _qkv(embeds, qw, kw, vw):
    return pl.pallas_call(
        _qkv_body,
        out_shape=(
            jax.ShapeDtypeStruct((N_NODES, D_MODEL), jnp.bfloat16),
            jax.ShapeDtypeStruct((N_NODES, 2 * D_MODEL), jnp.bfloat16),
        ),
    )(embeds, qw, kw, vw)


def _edge_body(q_hbm, kv_hbm, idx_hbm, nd_hbm, *sc):
    ib = sc[0:4]
    qb = sc[4:6]
    kvb = sc[6:8]
    wd = sc[8:10]
    zn = sc[10]
    nd_sh = sc[11]
    isem = sc[12:16]
    gsem = sc[16:18]
    ssem = sc[18:20]

    c = lax.axis_index("c")
    s = lax.axis_index("s")
    w = c * _NS + s
    zero16 = jnp.zeros((16,), jnp.float32)

    @pl.loop(0, _ZB)
    def _(r):
        @pl.loop(0, _D2 // 16)
        def _(cc):
            zn[r, pl.ds(cc * 16, 16)] = zero16

    base = s * _RPT

    @pl.loop(0, _RPT // _ZB)
    def _(j):
        pltpu.sync_copy(zn, nd_sh.at[pl.ds(base + j * _ZB, _ZB)])

    plsc.subcore_barrier()

    wblk = w * _NBLK

    def issue_idx(blk, sl):
        pltpu.async_copy(idx_hbm.at[blk], ib[sl], isem[sl])

    def wait_idx(sl):
        pltpu.make_async_copy(idx_hbm.at[0], ib[sl], isem[sl]).wait()

    def issue_gather(gs, isl, row):
        pltpu.async_copy(q_hbm.at[ib[isl].at[0, row]], qb[gs], gsem[gs])
        pltpu.async_copy(kv_hbm.at[ib[isl].at[1, row]], kvb[gs], gsem[gs])

    def wait_gather(gs):
        pltpu.make_async_copy(q_hbm.at[pl.ds(0, _CH)], qb[gs],
                              gsem[gs]).wait()
        pltpu.make_async_copy(kv_hbm.at[pl.ds(0, _CH)], kvb[gs],
                              gsem[gs]).wait()

    def compute(cs):
        qs, kvs, ws = qb[cs], kvb[cs], wd[cs]

        @plsc.parallel_loop(0, _CH, unroll=4)
        def _(e):
            p = []
            for g in range(4):
                qa, qo = plsc.unpack(qs[e, pl.ds(32 * g, 32)],
                                     format=plsc.PackFormat.INTERLEAVED)
                ka, ko = plsc.unpack(kvs[e, pl.ds(32 * g, 32)],
                                     format=plsc.PackFormat.INTERLEAVED)
                p.append(qa * ka)
                p.append(qo * ko)
            acc = ((p[0] + p[1]) + (p[2] + p[3])) + \
                  ((p[4] + p[5]) + (p[6] + p[7]))
            attv = acc + lax.rev(acc, (0,))
            attv = jnp.clip(attv, -10.0, 10.0)
            ev = jnp.exp(attv)
            ws[e, pl.ds(D_MODEL, 16)] = ev
            for g in range(4):
                va, vo = plsc.unpack(kvs[e, pl.ds(D_MODEL + 32 * g, 32)],
                                     format=plsc.PackFormat.INTERLEAVED)
                ws[e, pl.ds(32 * g, 16)] = va * ev
                ws[e, pl.ds(32 * g + 16, 16)] = vo * ev

    def issue_scatter(cs, isl, row):
        pltpu.async_copy(wd[cs], nd_sh.at[ib[isl].at[2, row]], ssem[cs],
                         add=True)

    def wait_scatter(cs):
        pltpu.make_async_copy(wd[cs], nd_sh.at[pl.ds(0, _CH)],
                              ssem[cs]).wait()

    issue_idx(wblk, 0)
    issue_idx(wblk + 1, 1)
    issue_idx(wblk + 2, 2)
    wait_idx(0)
    issue_gather(0, 0, 0)

    @pl.loop(0, _NT)
    def _(T):
        for j in range(_WIN):
            cs = j % 2

            def advance(j=j):
                if j % 2 == 1:
                    wait_idx(((j + 1) // 2) % 4)
                issue_gather((j + 1) % 2, ((j + 1) // 2) % 4, (j + 1) % 2)

            if j == _WIN - 1:
                @pl.when(T < _NT - 1)
                def _(advance=advance):
                    advance()
            else:
                advance()

            wait_gather(cs)

            if j < 2:
                @pl.when(T > 0)
                def _(cs=cs):
                    wait_scatter(cs)
            else:
                wait_scatter(cs)

            compute(cs)
            issue_scatter(cs, j // 2, j % 2)

            if j % 2 == 1:
                boff = (j + 5) // 2
                isl = boff % 4
                if j == 1:
                    issue_idx(wblk + 4 * T + boff, isl)
                else:
                    @pl.when(T < _NT - 1)
                    def _(boff=boff, isl=isl):
                        issue_idx(wblk + 4 * T + boff, isl)

    wait_scatter(0)
    wait_scatter(1)
    plsc.subcore_barrier()
    pltpu.sync_copy(nd_sh.at[pl.ds(base, _RPT)],
                    nd_hbm.at[c, pl.ds(base, _RPT)])


_edge_pass = pl.kernel(
    _edge_body,
    out_type=jax.ShapeDtypeStruct((_NC, _NPAD, _D2), jnp.float32),
    mesh=plsc.VectorSubcoreMesh(core_axis_name="c", subcore_axis_name="s"),
    compiler_params=pltpu.CompilerParams(use_tc_tiling_on_sc=False,
                                         needs_layout_passes=False),
    scratch_types=(
        [pltpu.VMEM((3, 2, _CH), jnp.int32)] * 4
        + [pltpu.VMEM((_CH, D_MODEL), jnp.bfloat16)] * 2
        + [pltpu.VMEM((_CH, 2 * D_MODEL), jnp.bfloat16)] * 2
        + [pltpu.VMEM((_CH, _D2), jnp.float32)] * 2
        + [
            pltpu.VMEM((_ZB, _D2), jnp.float32),
            pltpu.VMEM_SHARED((_NPAD, _D2), jnp.float32),
        ]
        + [pltpu.SemaphoreType.DMA] * 8
    ),
)


def _final_body(num_ref, den_ref, emb_ref, g_ref, b_ref, o_ref):
    num = num_ref[0] + num_ref[1]
    den = den_ref[0] + den_ref[1]
    den128 = pltpu.repeat(den, 8, axis=1)
    r = num / (den128 + 1e-8) + emb_ref[...]
    mean = jnp.mean(r, axis=-1, keepdims=True)
    cen = r - mean
    var = jnp.mean(cen * cen, axis=-1, keepdims=True)
    o_ref[...] = cen / jnp.sqrt(var + 1e-6) * g_ref[...] + b_ref[...]


def _finalize(num, den, embp, gp, bp):
    out = jax.ShapeDtypeStruct((N_NODES, D_MODEL), jnp.float32)
    blk = 1000
    return pl.pallas_call(
        _final_body,
        grid=(N_NODES // blk,),
        in_specs=[
            pl.BlockSpec((_NC, blk, D_MODEL), lambda i: (0, i, 0)),
            pl.BlockSpec((_NC, blk, 16), lambda i: (0, i, 0)),
            pl.BlockSpec((blk, D_MODEL), lambda i: (i, 0)),
            pl.BlockSpec((1, D_MODEL), lambda i: (0, 0)),
            pl.BlockSpec((1, D_MODEL), lambda i: (0, 0)),
        ],
        out_specs=pl.BlockSpec((blk, D_MODEL), lambda i: (i, 0)),
        out_shape=out,
    )(num, den, embp, gp, bp)


def kernel(embeds, edge_index, qTrans, kTrans, vTrans, ln_gamma, ln_beta):
    rows = edge_index[0].astype(jnp.int32)
    cols = edge_index[1].astype(jnp.int32)
    npad = _EPAD - N_EDGES
    zpad = jnp.zeros((npad,), jnp.int32)
    nblk_total = _EPAD // (2 * _CH)
    g_rows = jnp.concatenate([rows, zpad]).reshape(nblk_total, 2, _CH)
    g_cols = jnp.concatenate([cols, zpad]).reshape(nblk_total, 2, _CH)
    s_rows = jnp.concatenate(
        [rows, jnp.full((npad,), _NPAD - 2, jnp.int32)]
    ).reshape(nblk_total, 2, _CH)
    idx = jnp.stack([g_rows, g_cols, s_rows], axis=1)
    perm = jnp.asarray(_PERM)
    sperm = jnp.asarray(_SPERM)
    qp, kvp = _qkv(embeds, qTrans[:, perm], kTrans[:, perm], vTrans[:, perm])
    nd = _edge_pass(qp, kvp, idx)
    num = nd[:, :N_NODES, :D_MODEL]
    den = nd[:, :N_NODES, D_MODEL:]
    outp = _finalize(num, den, embeds[:, sperm],
                     ln_gamma[sperm].reshape(1, D_MODEL),
                     ln_beta[sperm].reshape(1, D_MODEL))
    return outp[:, jnp.asarray(_INVSPERM)]


# speedup vs baseline: 1.0414x; 1.0414x over previous
import functools

import jax
import jax.numpy as jnp
import numpy as np
from jax import lax
from jax.experimental import pallas as pl
from jax.experimental.pallas import tpu as pltpu
from jax.experimental.pallas import tpu_sc as plsc

N_NODES = 10000
N_EDGES = 320000
D_MODEL = 128
N_HEAD = 8

_NC = 2
_NS = 16
_NW = _NC * _NS
_CH = 40
_CPT = 256
_EPAD = _NW * _CPT * _CH
_NBLK = _CPT // 2
_WIN = 8
_NT = _CPT // _WIN
_D2 = D_MODEL + 16
_NPAD = 10240
_RPT = _NPAD // _NS
_ZB = 8

_PERM = np.zeros(D_MODEL, np.int32)
_SPERM = np.zeros(D_MODEL, np.int32)
for _m in range(8):
    _g, _par = divmod(_m, 2)
    for _l in range(16):
        _h = _l if _l < 8 else 15 - _l
        _d = 2 * _m if _l < 8 else 2 * _m + 1
        _oc = 16 * _h + _d
        _PERM[32 * _g + 2 * _l + _par] = _oc
        _SPERM[16 * _m + _l] = _oc
_INVSPERM = np.argsort(_SPERM)


def _qkv_body(emb_ref, qw_ref, kw_ref, vw_ref, qo_ref, kvo_ref):
    e = emb_ref[...]
    hi = lax.Precision.HIGHEST
    qo_ref[...] = jnp.dot(e, qw_ref[...], precision=hi).astype(jnp.bfloat16)
    kvo_ref[:, pl.ds(0, D_MODEL)] = jnp.dot(
        e, kw_ref[...], precision=hi).astype(jnp.bfloat16)
    kvo_ref[:, pl.ds(D_MODEL, D_MODEL)] = jnp.dot(
        e, vw_ref[...], precision=hi).astype(jnp.bfloat16)


def _qkv(embeds, qw, kw, vw):
    return pl.pallas_call(
        _qkv_body,
        out_shape=(
            jax.ShapeDtypeStruct((N_NODES, D_MODEL), jnp.bfloat16),
            jax.ShapeDtypeStruct((N_NODES, 2 * D_MODEL), jnp.bfloat16),
        ),
    )(embeds, qw, kw, vw)


def _edge_body(q_hbm, kv_hbm, idx_hbm, num_hbm, den_hbm, *sc):
    ib = sc[0:4]
    qb = sc[4:6]
    kvb = sc[6:8]
    wv = sc[8:10]
    ab = sc[10:12]
    zn = sc[12]
    zd = sc[13]
    num_sh = sc[14]
    den_sh = sc[15]
    isem = sc[16:20]
    gsem = sc[20:22]
    ssem = sc[22:24]

    c = lax.axis_index("c")
    s = lax.axis_index("s")
    w = c * _NS + s
    zero16 = jnp.zeros((16,), jnp.float32)

    @pl.loop(0, _ZB)
    def _(r):
        zd[r, :] = zero16

        @pl.loop(0, 8)
        def _(cc):
            zn[r, pl.ds(cc * 16, 16)] = zero16

    base = s * _RPT

    @pl.loop(0, _RPT // _ZB)
    def _(j):
        pltpu.sync_copy(zn, num_sh.at[pl.ds(base + j * _ZB, _ZB)])
        pltpu.sync_copy(zd, den_sh.at[pl.ds(base + j * _ZB, _ZB)])

    plsc.subcore_barrier()

    wblk = w * _NBLK

    def issue_idx(blk, sl):
        pltpu.async_copy(idx_hbm.at[blk], ib[sl], isem[sl])

    def wait_idx(sl):
        pltpu.make_async_copy(idx_hbm.at[0], ib[sl], isem[sl]).wait()

    def issue_gather(gs, isl, row):
        pltpu.async_copy(q_hbm.at[ib[isl].at[0, row]], qb[gs], gsem[gs])
        pltpu.async_copy(kv_hbm.at[ib[isl].at[1, row]], kvb[gs], gsem[gs])

    def wait_gather(gs):
        pltpu.make_async_copy(q_hbm.at[pl.ds(0, _CH)], qb[gs],
                              gsem[gs]).wait()
        pltpu.make_async_copy(kv_hbm.at[pl.ds(0, _CH)], kvb[gs],
                              gsem[gs]).wait()

    def compute(cs):
        qs, kvs, ws, as_ = qb[cs], kvb[cs], wv[cs], ab[cs]

        @plsc.parallel_loop(0, _CH, unroll=4)
        def _(e):
            p = []
            for g in range(4):
                qa, qo = plsc.unpack(qs[e, pl.ds(32 * g, 32)],
                                     format=plsc.PackFormat.INTERLEAVED)
                ka, ko = plsc.unpack(kvs[e, pl.ds(32 * g, 32)],
                                     format=plsc.PackFormat.INTERLEAVED)
                p.append(qa * ka)
                p.append(qo * ko)
            acc = ((p[0] + p[1]) + (p[2] + p[3])) + \
                  ((p[4] + p[5]) + (p[6] + p[7]))
            attv = acc + lax.rev(acc, (0,))
            attv = jnp.clip(attv, -10.0, 10.0)
            ev = jnp.exp(attv)
            as_[e, :] = ev
            for g in range(4):
                va, vo = plsc.unpack(kvs[e, pl.ds(D_MODEL + 32 * g, 32)],
                                     format=plsc.PackFormat.INTERLEAVED)
                ws[e, pl.ds(32 * g, 16)] = va * ev
                ws[e, pl.ds(32 * g + 16, 16)] = vo * ev

    def issue_scatter(cs, isl, row):
        pltpu.async_copy(wv[cs], num_sh.at[ib[isl].at[2, row]], ssem[cs],
                         add=True)
        pltpu.async_copy(ab[cs], den_sh.at[ib[isl].at[2, row]], ssem[cs],
                         add=True)

    def wait_scatter(cs):
        pltpu.make_async_copy(wv[cs], num_sh.at[pl.ds(0, _CH)],
                              ssem[cs]).wait()
        pltpu.make_async_copy(ab[cs], den_sh.at[pl.ds(0, _CH)],
                              ssem[cs]).wait()

    issue_idx(wblk, 0)
    issue_idx(wblk + 1, 1)
    issue_idx(wblk + 2, 2)
    wait_idx(0)
    issue_gather(0, 0, 0)

    @pl.loop(0, _NT)
    def _(T):
        for j in range(_WIN):
            cs = j % 2

            def advance(j=j):
                if j % 2 == 1:
                    wait_idx(((j + 1) // 2) % 4)
                issue_gather((j + 1) % 2, ((j + 1) // 2) % 4, (j + 1) % 2)

            if j == _WIN - 1:
                @pl.when(T < _NT - 1)
                def _(advance=advance):
                    advance()
            else:
                advance()

            wait_gather(cs)

            if j < 2:
                @pl.when(T > 0)
                def _(cs=cs):
                    wait_scatter(cs)
            else:
                wait_scatter(cs)

            compute(cs)
            issue_scatter(cs, j // 2, j % 2)

            if j % 2 == 1:
                boff = (j + 5) // 2
                isl = boff % 4
                if j == 1:
                    issue_idx(wblk + 4 * T + boff, isl)
                else:
                    @pl.when(T < _NT - 1)
                    def _(boff=boff, isl=isl):
                        issue_idx(wblk + 4 * T + boff, isl)

    wait_scatter(0)
    wait_scatter(1)
    plsc.subcore_barrier()
    pltpu.sync_copy(num_sh.at[pl.ds(base, _RPT)],
                    num_hbm.at[c, pl.ds(base, _RPT)])
    pltpu.sync_copy(den_sh.at[pl.ds(base, _RPT)],
                    den_hbm.at[c, pl.ds(base, _RPT)])


_edge_pass = pl.kernel(
    _edge_body,
    out_type=(
        jax.ShapeDtypeStruct((_NC, _NPAD, D_MODEL), jnp.float32),
        jax.ShapeDtypeStruct((_NC, _NPAD, 16), jnp.float32),
    ),
    mesh=plsc.VectorSubcoreMesh(core_axis_name="c", subcore_axis_name="s"),
    compiler_params=pltpu.CompilerParams(use_tc_tiling_on_sc=False,
                                         needs_layout_passes=False),
    scratch_types=(
        [pltpu.VMEM((3, 2, _CH), jnp.int32)] * 4
        + [pltpu.VMEM((_CH, D_MODEL), jnp.bfloat16)] * 2
        + [pltpu.VMEM((_CH, 2 * D_MODEL), jnp.bfloat16)] * 2
        + [pltpu.VMEM((_CH, D_MODEL), jnp.float32)] * 2
        + [pltpu.VMEM((_CH, 16), jnp.float32)] * 2
        + [
            pltpu.VMEM((_ZB, D_MODEL), jnp.float32),
            pltpu.VMEM((_ZB, 16), jnp.float32),
            pltpu.VMEM_SHARED((_NPAD, D_MODEL), jnp.float32),
            pltpu.VMEM_SHARED((_NPAD, 16), jnp.float32),
        ]
        + [pltpu.SemaphoreType.DMA] * 8
    ),
)


def _final_body(num_ref, den_ref, emb_ref, g_ref, b_ref, o_ref):
    num = num_ref[0] + num_ref[1]
    den = den_ref[0] + den_ref[1]
    den128 = pltpu.repeat(den, 8, axis=1)
    r = num / (den128 + 1e-8) + emb_ref[...]
    mean = jnp.mean(r, axis=-1, keepdims=True)
    cen = r - mean
    var = jnp.mean(cen * cen, axis=-1, keepdims=True)
    o_ref[...] = cen / jnp.sqrt(var + 1e-6) * g_ref[...] + b_ref[...]


def _finalize(num, den, embp, gp, bp):
    out = jax.ShapeDtypeStruct((N_NODES, D_MODEL), jnp.float32)
    blk = 1000
    return pl.pallas_call(
        _final_body,
        grid=(N_NODES // blk,),
        in_specs=[
            pl.BlockSpec((_NC, blk, D_MODEL), lambda i: (0, i, 0)),
            pl.BlockSpec((_NC, blk, 16), lambda i: (0, i, 0)),
            pl.BlockSpec((blk, D_MODEL), lambda i: (i, 0)),
            pl.BlockSpec((1, D_MODEL), lambda i: (0, 0)),
            pl.BlockSpec((1, D_MODEL), lambda i: (0, 0)),
        ],
        out_specs=pl.BlockSpec((blk, D_MODEL), lambda i: (i, 0)),
        out_shape=out,
    )(num, den, embp, gp, bp)


def kernel(embeds, edge_index, qTrans, kTrans, vTrans, ln_gamma, ln_beta):
    rows = edge_index[0].astype(jnp.int32)
    cols = edge_index[1].astype(jnp.int32)
    npad = _EPAD - N_EDGES
    zpad = jnp.zeros((npad,), jnp.int32)
    nblk_total = _EPAD // (2 * _CH)
    g_rows = jnp.concatenate([rows, zpad]).reshape(nblk_total, 2, _CH)
    g_cols = jnp.concatenate([cols, zpad]).reshape(nblk_total, 2, _CH)
    s_rows = jnp.concatenate(
        [rows, jnp.full((npad,), _NPAD - 2, jnp.int32)]
    ).reshape(nblk_total, 2, _CH)
    idx = jnp.stack([g_rows, g_cols, s_rows], axis=1)
    perm = jnp.asarray(_PERM)
    sperm = jnp.asarray(_SPERM)
    qp, kvp = _qkv(embeds, qTrans[:, perm], kTrans[:, perm], vTrans[:, perm])
    num, den = _edge_pass(qp, kvp, idx)
    num = num[:, :N_NODES]
    den = den[:, :N_NODES]
    outp = _finalize(num, den, embeds[:, sperm],
                     ln_gamma[sperm].reshape(1, D_MODEL),
                     ln_beta[sperm].reshape(1, D_MODEL))
    return outp[:, jnp.asarray(_INVSPERM)]


# speedup vs baseline: 1.2918x; 1.2405x over previous
import functools

import jax
import jax.numpy as jnp
import numpy as np
from jax import lax
from jax.experimental import pallas as pl
from jax.experimental.pallas import tpu as pltpu
from jax.experimental.pallas import tpu_sc as plsc

N_NODES = 10000
N_EDGES = 320000
D_MODEL = 128
N_HEAD = 8

_NC = 2
_NS = 16
_NW = _NC * _NS
_CH = 40
_CPT = 256
_EPAD = _NW * _CPT * _CH
_NBLK = _CPT // 2
_WIN = 8
_NT = _CPT // _WIN
_D2 = D_MODEL + 16
_NPAD = 10240
_RPT = _NPAD // _NS
_ZB = 8

_PERM = np.zeros(D_MODEL, np.int32)
_SPERM = np.zeros(D_MODEL, np.int32)
for _m in range(8):
    _g, _par = divmod(_m, 2)
    for _l in range(16):
        _h = _l if _l < 8 else 15 - _l
        _d = 2 * _m if _l < 8 else 2 * _m + 1
        _oc = 16 * _h + _d
        _PERM[32 * _g + 2 * _l + _par] = _oc
        _SPERM[16 * _m + _l] = _oc
_INVSPERM = np.argsort(_SPERM)


def _qkv_body(emb_ref, qw_ref, kw_ref, vw_ref, qo_ref, ko_ref, vo_ref):
    e = emb_ref[...]
    hi = lax.Precision.HIGHEST
    qo_ref[...] = jnp.dot(e, qw_ref[...], precision=hi).astype(jnp.bfloat16)
    ko_ref[...] = jnp.dot(e, kw_ref[...], precision=hi).astype(jnp.bfloat16)
    vo_ref[...] = jnp.dot(e, vw_ref[...], precision=hi).astype(jnp.bfloat16)


def _qkv(embeds, qw, kw, vw):
    out = jax.ShapeDtypeStruct((N_NODES, D_MODEL), jnp.bfloat16)
    return pl.pallas_call(_qkv_body, out_shape=(out, out, out))(
        embeds, qw, kw, vw)


def _edge_body(q_hbm, k_hbm, v_hbm, gr_hbm, gc_hbm, sr_hbm, num_hbm, den_hbm,
               *sc):
    gr = sc[0:4]
    gc = sc[4:8]
    sr = sc[8:12]
    qb = sc[12:14]
    kb = sc[14:16]
    vb = sc[16:18]
    wv = sc[18:20]
    ab = sc[20:22]
    zn = sc[22]
    zd = sc[23]
    num_sh = sc[24]
    den_sh = sc[25]
    isem = sc[26:30]
    gsem = sc[30:32]
    ssem = sc[32:34]

    c = lax.axis_index("c")
    s = lax.axis_index("s")
    w = c * _NS + s
    zero16 = jnp.zeros((16,), jnp.float32)

    @pl.loop(0, _ZB)
    def _(r):
        zd[r, :] = zero16

        @pl.loop(0, 8)
        def _(cc):
            zn[r, pl.ds(cc * 16, 16)] = zero16

    base = s * _RPT

    @pl.loop(0, _RPT // _ZB)
    def _(j):
        pltpu.sync_copy(zn, num_sh.at[pl.ds(base + j * _ZB, _ZB)])
        pltpu.sync_copy(zd, den_sh.at[pl.ds(base + j * _ZB, _ZB)])

    plsc.subcore_barrier()

    wblk = w * _NBLK

    def issue_idx(blk, sl):
        pltpu.async_copy(gr_hbm.at[blk], gr[sl], isem[sl])
        pltpu.async_copy(gc_hbm.at[blk], gc[sl], isem[sl])
        pltpu.async_copy(sr_hbm.at[blk], sr[sl], isem[sl])

    def wait_idx(sl):
        for ref in (gr[sl], gc[sl], sr[sl]):
            pltpu.make_async_copy(gr_hbm.at[0], ref, isem[sl]).wait()

    def issue_gather(gs, isl, row):
        pltpu.async_copy(q_hbm.at[gr[isl].at[row]], qb[gs], gsem[gs])
        pltpu.async_copy(k_hbm.at[gc[isl].at[row]], kb[gs], gsem[gs])
        pltpu.async_copy(v_hbm.at[gc[isl].at[row]], vb[gs], gsem[gs])

    def wait_gather(gs):
        pltpu.make_async_copy(q_hbm.at[pl.ds(0, _CH)], qb[gs],
                              gsem[gs]).wait()
        pltpu.make_async_copy(k_hbm.at[pl.ds(0, _CH)], kb[gs],
                              gsem[gs]).wait()
        pltpu.make_async_copy(v_hbm.at[pl.ds(0, _CH)], vb[gs],
                              gsem[gs]).wait()

    def compute(cs):
        qs, ks, vs, ws, as_ = qb[cs], kb[cs], vb[cs], wv[cs], ab[cs]

        @plsc.parallel_loop(0, _CH, unroll=4)
        def _(e):
            p = []
            for g in range(4):
                qa, qo = plsc.unpack(qs[e, pl.ds(32 * g, 32)],
                                     format=plsc.PackFormat.INTERLEAVED)
                ka, ko = plsc.unpack(ks[e, pl.ds(32 * g, 32)],
                                     format=plsc.PackFormat.INTERLEAVED)
                p.append(qa * ka)
                p.append(qo * ko)
            acc = ((p[0] + p[1]) + (p[2] + p[3])) + \
                  ((p[4] + p[5]) + (p[6] + p[7]))
            attv = acc + lax.rev(acc, (0,))
            attv = jnp.clip(attv, -10.0, 10.0)
            ev = jnp.exp(attv)
            as_[e, :] = ev
            for g in range(4):
                va, vo = plsc.unpack(vs[e, pl.ds(32 * g, 32)],
                                     format=plsc.PackFormat.INTERLEAVED)
                ws[e, pl.ds(32 * g, 16)] = va * ev
                ws[e, pl.ds(32 * g + 16, 16)] = vo * ev

    def issue_scatter(cs, isl, row):
        pltpu.async_copy(wv[cs], num_sh.at[sr[isl].at[row]], ssem[cs],
                         add=True)
        pltpu.async_copy(ab[cs], den_sh.at[sr[isl].at[row]], ssem[cs],
                         add=True)

    def wait_scatter(cs):
        pltpu.make_async_copy(wv[cs], num_sh.at[pl.ds(0, _CH)],
                              ssem[cs]).wait()
        pltpu.make_async_copy(ab[cs], den_sh.at[pl.ds(0, _CH)],
                              ssem[cs]).wait()

    issue_idx(wblk, 0)
    issue_idx(wblk + 1, 1)
    issue_idx(wblk + 2, 2)
    wait_idx(0)
    issue_gather(0, 0, 0)

    @pl.loop(0, _NT)
    def _(T):
        for j in range(_WIN):
            cs = j % 2

            def advance(j=j):
                if j % 2 == 1:
                    wait_idx(((j + 1) // 2) % 4)
                issue_gather((j + 1) % 2, ((j + 1) // 2) % 4, (j + 1) % 2)

            if j == _WIN - 1:
                @pl.when(T < _NT - 1)
                def _(advance=advance):
                    advance()
            else:
                advance()

            wait_gather(cs)

            if j < 2:
                @pl.when(T > 0)
                def _(cs=cs):
                    wait_scatter(cs)
            else:
                wait_scatter(cs)

            compute(cs)
            issue_scatter(cs, j // 2, j % 2)

            if j % 2 == 1:
                boff = (j + 5) // 2
                isl = boff % 4
                if j == 1:
                    issue_idx(wblk + 4 * T + boff, isl)
                else:
                    @pl.when(T < _NT - 1)
                    def _(boff=boff, isl=isl):
                        issue_idx(wblk + 4 * T + boff, isl)

    wait_scatter(0)
    wait_scatter(1)
    plsc.subcore_barrier()
    pltpu.sync_copy(num_sh.at[pl.ds(base, _RPT)],
                    num_hbm.at[c, pl.ds(base, _RPT)])
    pltpu.sync_copy(den_sh.at[pl.ds(base, _RPT)],
                    den_hbm.at[c, pl.ds(base, _RPT)])


_edge_pass = pl.kernel(
    _edge_body,
    out_type=(
        jax.ShapeDtypeStruct((_NC, _NPAD, D_MODEL), jnp.float32),
        jax.ShapeDtypeStruct((_NC, _NPAD, 16), jnp.float32),
    ),
    mesh=plsc.VectorSubcoreMesh(core_axis_name="c", subcore_axis_name="s"),
    compiler_params=pltpu.CompilerParams(use_tc_tiling_on_sc=False,
                                         needs_layout_passes=False),
    scratch_types=(
        [pltpu.VMEM((2, _CH), jnp.int32)] * 12
        + [pltpu.VMEM((_CH, D_MODEL), jnp.bfloat16)] * 6
        + [pltpu.VMEM((_CH, D_MODEL), jnp.float32)] * 2
        + [pltpu.VMEM((_CH, 16), jnp.float32)] * 2
        + [
            pltpu.VMEM((_ZB, D_MODEL), jnp.float32),
            pltpu.VMEM((_ZB, 16), jnp.float32),
            pltpu.VMEM_SHARED((_NPAD, D_MODEL), jnp.float32),
            pltpu.VMEM_SHARED((_NPAD, 16), jnp.float32),
        ]
        + [pltpu.SemaphoreType.DMA] * 8
    ),
)


def _final_body(num_ref, den_ref, emb_ref, g_ref, b_ref, o_ref):
    num = num_ref[0] + num_ref[1]
    den = den_ref[0] + den_ref[1]
    den128 = pltpu.repeat(den, 8, axis=1)
    r = num / (den128 + 1e-8) + emb_ref[...]
    mean = jnp.mean(r, axis=-1, keepdims=True)
    cen = r - mean
    var = jnp.mean(cen * cen, axis=-1, keepdims=True)
    o_ref[...] = cen / jnp.sqrt(var + 1e-6) * g_ref[...] + b_ref[...]


def _finalize(num, den, embp, gp, bp):
    out = jax.ShapeDtypeStruct((N_NODES, D_MODEL), jnp.float32)
    blk = 1000
    return pl.pallas_call(
        _final_body,
        grid=(N_NODES // blk,),
        in_specs=[
            pl.BlockSpec((_NC, blk, D_MODEL), lambda i: (0, i, 0)),
            pl.BlockSpec((_NC, blk, 16), lambda i: (0, i, 0)),
            pl.BlockSpec((blk, D_MODEL), lambda i: (i, 0)),
            pl.BlockSpec((1, D_MODEL), lambda i: (0, 0)),
            pl.BlockSpec((1, D_MODEL), lambda i: (0, 0)),
        ],
        out_specs=pl.BlockSpec((blk, D_MODEL), lambda i: (i, 0)),
        out_shape=out,
    )(num, den, embp, gp, bp)


def kernel(embeds, edge_index, qTrans, kTrans, vTrans, ln_gamma, ln_beta):
    rows = edge_index[0].astype(jnp.int32)
    cols = edge_index[1].astype(jnp.int32)
    npad = _EPAD - N_EDGES
    zpad = jnp.zeros((npad,), jnp.int32)
    nblk_total = _EPAD // (2 * _CH)
    g_rows = jnp.concatenate([rows, zpad]).reshape(nblk_total, 2, _CH)
    g_cols = jnp.concatenate([cols, zpad]).reshape(nblk_total, 2, _CH)
    s_rows = jnp.concatenate(
        [rows, jnp.full((npad,), _NPAD - 2, jnp.int32)]
    ).reshape(nblk_total, 2, _CH)
    perm = jnp.asarray(_PERM)
    sperm = jnp.asarray(_SPERM)
    qp, kp, vp = _qkv(embeds, qTrans[:, perm], kTrans[:, perm],
                      vTrans[:, perm])
    num, den = _edge_pass(qp, kp, vp, g_rows, g_cols, s_rows)
    num = num[:, :N_NODES]
    den = den[:, :N_NODES]
    outp = _finalize(num, den, embeds[:, sperm],
                     ln_gamma[sperm].reshape(1, D_MODEL),
                     ln_beta[sperm].reshape(1, D_MODEL))
    return outp[:, jnp.asarray(_INVSPERM)]


# speedup vs baseline: 1.5258x; 1.1811x over previous
import functools

import jax
import jax.numpy as jnp
import numpy as np
from jax import lax
from jax.experimental import pallas as pl
from jax.experimental.pallas import tpu as pltpu
from jax.experimental.pallas import tpu_sc as plsc

N_NODES = 10000
N_EDGES = 320000
D_MODEL = 128
N_HEAD = 8

_NC = 2
_NS = 16
_NW = _NC * _NS
_CH = 40
_CPT = 256
_EPAD = _NW * _CPT * _CH
_NBLK = _CPT // 2
_WIN = 8
_NT = _CPT // _WIN
_D2 = D_MODEL + 16
_NPAD = 10240
_RPT = _NPAD // _NS
_ZB = 8

_PERM = np.zeros(D_MODEL, np.int32)
_SPERM = np.zeros(D_MODEL, np.int32)
for _m in range(8):
    _g, _par = divmod(_m, 2)
    for _l in range(16):
        _h = _l if _l < 8 else 15 - _l
        _d = 2 * _m if _l < 8 else 2 * _m + 1
        _oc = 16 * _h + _d
        _PERM[32 * _g + 2 * _l + _par] = _oc
        _SPERM[16 * _m + _l] = _oc
_INVSPERM = np.argsort(_SPERM)


def _qkv_body(emb_ref, qw_ref, kw_ref, vw_ref, qo_ref, ko_ref, vo_ref):
    e = emb_ref[...]
    hi = lax.Precision.HIGHEST
    qo_ref[...] = jnp.dot(e, qw_ref[...], precision=hi).astype(jnp.bfloat16)
    ko_ref[...] = jnp.dot(e, kw_ref[...], precision=hi).astype(jnp.bfloat16)
    vo_ref[...] = jnp.dot(e, vw_ref[...], precision=hi).astype(jnp.bfloat16)


def _qkv(embeds, qw, kw, vw):
    out = jax.ShapeDtypeStruct((N_NODES, D_MODEL), jnp.bfloat16)
    return pl.pallas_call(_qkv_body, out_shape=(out, out, out))(
        embeds, qw, kw, vw)


def _edge_body(q_hbm, k_hbm, v_hbm, gr_hbm, gc_hbm, sr_hbm, num_hbm, den_hbm,
               *sc):
    gr = sc[0:4]
    gc = sc[4:8]
    sr = sc[8:12]
    qb = sc[12:14]
    kb = sc[14:16]
    vb = sc[16:18]
    wv = sc[18:20]
    ab = sc[20:22]
    zn = sc[22]
    zd = sc[23]
    num_sh = sc[24]
    den_sh = sc[25]
    isem = sc[26:30]
    gsem = sc[30:32]
    ssem = sc[32:34]

    c = lax.axis_index("c")
    s = lax.axis_index("s")
    w = c * _NS + s
    zero16 = jnp.zeros((16,), jnp.float32)

    @pl.loop(0, _ZB)
    def _(r):
        zd[r, :] = zero16

        @pl.loop(0, 8)
        def _(cc):
            zn[r, pl.ds(cc * 16, 16)] = zero16

    base = s * _RPT

    @pl.loop(0, _RPT // _ZB)
    def _(j):
        pltpu.sync_copy(zn, num_sh.at[pl.ds(base + j * _ZB, _ZB)])
        pltpu.sync_copy(zd, den_sh.at[pl.ds(base + j * _ZB, _ZB)])

    plsc.subcore_barrier()

    wblk = w * _NBLK

    def issue_idx(blk, sl):
        pltpu.async_copy(gr_hbm.at[blk], gr[sl], isem[sl])
        pltpu.async_copy(gc_hbm.at[blk], gc[sl], isem[sl])
        pltpu.async_copy(sr_hbm.at[blk], sr[sl], isem[sl])

    def wait_idx(sl):
        for ref in (gr[sl], gc[sl], sr[sl]):
            pltpu.make_async_copy(gr_hbm.at[0], ref, isem[sl]).wait()

    def issue_gather(gs, isl, row):
        pltpu.async_copy(q_hbm.at[gr[isl].at[row]], qb[gs], gsem[gs])
        pltpu.async_copy(k_hbm.at[gc[isl].at[row]], kb[gs], gsem[gs])
        pltpu.async_copy(v_hbm.at[gc[isl].at[row]], vb[gs], gsem[gs])

    def wait_gather(gs):
        pltpu.make_async_copy(q_hbm.at[pl.ds(0, _CH)], qb[gs],
                              gsem[gs]).wait()
        pltpu.make_async_copy(k_hbm.at[pl.ds(0, _CH)], kb[gs],
                              gsem[gs]).wait()
        pltpu.make_async_copy(v_hbm.at[pl.ds(0, _CH)], vb[gs],
                              gsem[gs]).wait()

    def compute(cs):
        qs, ks, vs, ws, as_ = qb[cs], kb[cs], vb[cs], wv[cs], ab[cs]

        @plsc.parallel_loop(0, _CH, unroll=4)
        def _(e):
            p = []
            for g in range(4):
                prod = qs[e, pl.ds(32 * g, 32)] * ks[e, pl.ds(32 * g, 32)]
                pa, po = plsc.unpack(prod,
                                     format=plsc.PackFormat.INTERLEAVED)
                p.append(pa)
                p.append(po)
            acc = ((p[0] + p[1]) + (p[2] + p[3])) + \
                  ((p[4] + p[5]) + (p[6] + p[7]))
            attv = acc + lax.rev(acc, (0,))
            attv = jnp.clip(attv, -10.0, 10.0)
            ev = jnp.exp(attv)
            as_[e, :] = ev
            for g in range(4):
                va, vo = plsc.unpack(vs[e, pl.ds(32 * g, 32)],
                                     format=plsc.PackFormat.INTERLEAVED)
                ws[e, pl.ds(32 * g, 16)] = va * ev
                ws[e, pl.ds(32 * g + 16, 16)] = vo * ev

    def issue_scatter(cs, isl, row):
        pltpu.async_copy(wv[cs], num_sh.at[sr[isl].at[row]], ssem[cs],
                         add=True)
        pltpu.async_copy(ab[cs], den_sh.at[sr[isl].at[row]], ssem[cs],
                         add=True)

    def wait_scatter(cs):
        pltpu.make_async_copy(wv[cs], num_sh.at[pl.ds(0, _CH)],
                              ssem[cs]).wait()
        pltpu.make_async_copy(ab[cs], den_sh.at[pl.ds(0, _CH)],
                              ssem[cs]).wait()

    issue_idx(wblk, 0)
    issue_idx(wblk + 1, 1)
    issue_idx(wblk + 2, 2)
    wait_idx(0)
    issue_gather(0, 0, 0)

    @pl.loop(0, _NT)
    def _(T):
        for j in range(_WIN):
            cs = j % 2

            def advance(j=j):
                if j % 2 == 1:
                    wait_idx(((j + 1) // 2) % 4)
                issue_gather((j + 1) % 2, ((j + 1) // 2) % 4, (j + 1) % 2)

            if j == _WIN - 1:
                @pl.when(T < _NT - 1)
                def _(advance=advance):
                    advance()
            else:
                advance()

            wait_gather(cs)

            if j < 2:
                @pl.when(T > 0)
                def _(cs=cs):
                    wait_scatter(cs)
            else:
                wait_scatter(cs)

            compute(cs)
            issue_scatter(cs, j // 2, j % 2)

            if j % 2 == 1:
                boff = (j + 5) // 2
                isl = boff % 4
                if j == 1:
                    issue_idx(wblk + 4 * T + boff, isl)
                else:
                    @pl.when(T < _NT - 1)
                    def _(boff=boff, isl=isl):
                        issue_idx(wblk + 4 * T + boff, isl)

    wait_scatter(0)
    wait_scatter(1)
    plsc.subcore_barrier()
    pltpu.sync_copy(num_sh.at[pl.ds(base, _RPT)],
                    num_hbm.at[c, pl.ds(base, _RPT)])
    pltpu.sync_copy(den_sh.at[pl.ds(base, _RPT)],
                    den_hbm.at[c, pl.ds(base, _RPT)])


_edge_pass = pl.kernel(
    _edge_body,
    out_type=(
        jax.ShapeDtypeStruct((_NC, _NPAD, D_MODEL), jnp.float32),
        jax.ShapeDtypeStruct((_NC, _NPAD, 16), jnp.float32),
    ),
    mesh=plsc.VectorSubcoreMesh(core_axis_name="c", subcore_axis_name="s"),
    compiler_params=pltpu.CompilerParams(use_tc_tiling_on_sc=False,
                                         needs_layout_passes=False),
    scratch_types=(
        [pltpu.VMEM((2, _CH), jnp.int32)] * 12
        + [pltpu.VMEM((_CH, D_MODEL), jnp.bfloat16)] * 6
        + [pltpu.VMEM((_CH, D_MODEL), jnp.float32)] * 2
        + [pltpu.VMEM((_CH, 16), jnp.float32)] * 2
        + [
            pltpu.VMEM((_ZB, D_MODEL), jnp.float32),
            pltpu.VMEM((_ZB, 16), jnp.float32),
            pltpu.VMEM_SHARED((_NPAD, D_MODEL), jnp.float32),
            pltpu.VMEM_SHARED((_NPAD, 16), jnp.float32),
        ]
        + [pltpu.SemaphoreType.DMA] * 8
    ),
)


def _final_body(num_ref, den_ref, emb_ref, pinv_ref, g_ref, b_ref, o_ref):
    num = num_ref[0] + num_ref[1]
    den = den_ref[0] + den_ref[1]
    den128 = pltpu.repeat(den, 8, axis=1)
    r = num / (den128 + 1e-8) + emb_ref[...]
    mean = jnp.mean(r, axis=-1, keepdims=True)
    cen = r - mean
    var = jnp.mean(cen * cen, axis=-1, keepdims=True)
    normed = cen / jnp.sqrt(var + 1e-6)
    unperm = jnp.dot(normed, pinv_ref[...], precision=lax.Precision.HIGHEST)
    o_ref[...] = unperm * g_ref[...] + b_ref[...]


def _finalize(num, den, embp, pinv, gp, bp):
    out = jax.ShapeDtypeStruct((N_NODES, D_MODEL), jnp.float32)
    blk = 1000
    return pl.pallas_call(
        _final_body,
        grid=(N_NODES // blk,),
        in_specs=[
            pl.BlockSpec((_NC, blk, D_MODEL), lambda i: (0, i, 0)),
            pl.BlockSpec((_NC, blk, 16), lambda i: (0, i, 0)),
            pl.BlockSpec((blk, D_MODEL), lambda i: (i, 0)),
            pl.BlockSpec((D_MODEL, D_MODEL), lambda i: (0, 0)),
            pl.BlockSpec((1, D_MODEL), lambda i: (0, 0)),
            pl.BlockSpec((1, D_MODEL), lambda i: (0, 0)),
        ],
        out_specs=pl.BlockSpec((blk, D_MODEL), lambda i: (i, 0)),
        out_shape=out,
    )(num, den, embp, pinv, gp, bp)


def kernel(embeds, edge_index, qTrans, kTrans, vTrans, ln_gamma, ln_beta):
    rows = edge_index[0].astype(jnp.int32)
    cols = edge_index[1].astype(jnp.int32)
    npad = _EPAD - N_EDGES
    zpad = jnp.zeros((npad,), jnp.int32)
    nblk_total = _EPAD // (2 * _CH)
    g_rows = jnp.concatenate([rows, zpad]).reshape(nblk_total, 2, _CH)
    g_cols = jnp.concatenate([cols, zpad]).reshape(nblk_total, 2, _CH)
    s_rows = jnp.concatenate(
        [rows, jnp.full((npad,), _NPAD - 2, jnp.int32)]
    ).reshape(nblk_total, 2, _CH)
    perm = jnp.asarray(_PERM)
    sperm = jnp.asarray(_SPERM)
    qp, kp, vp = _qkv(embeds, qTrans[:, perm], kTrans[:, perm],
                      vTrans[:, perm])
    num, den = _edge_pass(qp, kp, vp, g_rows, g_cols, s_rows)
    pinv = np.zeros((D_MODEL, D_MODEL), np.float32)
    pinv[np.arange(D_MODEL), _SPERM] = 1.0
    return _finalize(num, den, embeds[:, sperm], jnp.asarray(pinv),
                     ln_gamma.reshape(1, D_MODEL),
                     ln_beta.reshape(1, D_MODEL))
